# baseline (speedup 1.0000x reference)
import functools

import jax
import jax.numpy as jnp
from jax import lax
from jax.experimental import pallas as pl
from jax.experimental.pallas import tpu as pltpu
from jax.experimental.pallas import tpu_sc as plsc

NC = 2
NS = 16
L = 16
AW = 128
DR = 80


def _build_edge_aggregate(n_table, n_dst, epw, chunk, compute_q):
    assert epw % chunk == 0 and chunk % L == 0
    n_chunks = epw // chunk
    rows_per_tile = n_dst // NS
    half = 5056
    mesh = plsc.VectorSubcoreMesh(core_axis_name="c", subcore_axis_name="s",
                                  num_cores=NC, num_subcores=NS)

    scratch = dict(
        srcbuf=pltpu.VMEM((chunk,), jnp.int32),
        dstraw=pltpu.VMEM((chunk,), jnp.int32),
        dstbuf=pltpu.VMEM((chunk,), jnp.int32),
        rows_v=pltpu.VMEM((chunk, 128), jnp.float32),
        scaled_v=pltpu.VMEM((chunk, AW), jnp.float32),
        den_loc=pltpu.VMEM((DR, 128), jnp.float32),
        acc_sh=pltpu.VMEM_SHARED((n_dst, AW), jnp.float32),
        sem=pltpu.SemaphoreType.DMA,
    )
    if compute_q:
        scratch.update(
            asrc_v=pltpu.VMEM((n_table,), jnp.float32),
            adst_v=pltpu.VMEM((n_table,), jnp.float32),
            m_v=pltpu.VMEM((n_table,), jnp.float32),
            qbuf=pltpu.VMEM((chunk + L,), jnp.float32),
        )
    scratch_names = list(scratch.keys())

    def body(*refs):
        if compute_q:
            (h_hbm, asrc_hbm, adst_hbm, m_hbm, src_hbm, dst_hbm, zeros_hbm,
             out_hbm, den_hbm) = refs[:9]
        else:
            h_hbm, src_hbm, dst_hbm, zeros_hbm, out_hbm, den_hbm = refs[:6]
        sc = dict(zip(scratch_names, refs[-len(scratch_names):]))

        cid = lax.axis_index("c")
        sid = lax.axis_index("s")
        nbase = cid * half

        assert rows_per_tile % DR == 0
        for k in range(rows_per_tile // DR):
            pltpu.sync_copy(
                zeros_hbm,
                sc["acc_sh"].at[pl.ds(sid * rows_per_tile + k * DR, DR)])
        pltpu.sync_copy(zeros_hbm, sc["den_loc"])
        base_e = sid * epw
        if compute_q:
            pltpu.sync_copy(asrc_hbm, sc["asrc_v"])
            pltpu.sync_copy(adst_hbm, sc["adst_v"])
            pltpu.sync_copy(m_hbm, sc["m_v"])
        plsc.subcore_barrier()

        ones16 = jnp.full((L,), 1.0, jnp.float32)

        def chunk_body(it, carry):
            cbase = base_e + it * chunk
            pltpu.sync_copy(src_hbm.at[pl.ds(cbase, chunk)], sc["srcbuf"])
            pltpu.sync_copy(dst_hbm.at[pl.ds(cbase, chunk)], sc["dstraw"])
            for i in range(chunk // L):
                s16 = sc["srcbuf"][pl.ds(i * L, L)]
                d16 = sc["dstraw"][pl.ds(i * L, L)]
                loc = d16 - nbase
                inrange = (loc >= 0) & (loc < half)
                sc["dstbuf"][pl.ds(i * L, L)] = jnp.where(
                    inrange, loc, jnp.int32(half))
                if compute_q:
                    av = plsc.load_gather(sc["asrc_v"], [s16])
                    bv = plsc.load_gather(sc["adst_v"], [d16])
                    mv = plsc.load_gather(sc["m_v"], [d16])
                    e = av + bv
                    e = jnp.where(e >= 0, e, e * jnp.float32(0.2))
                    q16 = jnp.exp(e - mv)
                    sc["qbuf"][pl.ds(L + i * L, L)] = q16
                else:
                    q16 = ones16
                drow = lax.shift_right_logical(d16, 7)
                dcol = lax.bitwise_and(d16, jnp.int32(127))
                plsc.addupdate_scatter(sc["den_loc"], [drow, dcol], q16)
            pltpu.async_copy(h_hbm.at[sc["srcbuf"]], sc["rows_v"], sc["sem"]).wait()
            if compute_q:
                for j in range(chunk):
                    qb = plsc.load_gather(sc["qbuf"],
                                          [jnp.full((L,), L + j, jnp.int32)])
                    for k in range(8):
                        sc["scaled_v"][j, pl.ds(k * L, L)] = (
                            sc["rows_v"][j, pl.ds(k * L, L)] * qb)
            src_rows = sc["scaled_v"] if compute_q else sc["rows_v"]
            pltpu.sync_copy(src_rows, sc["acc_sh"].at[sc["dstbuf"]], add=True)
            return carry

        lax.fori_loop(0, n_chunks, chunk_body, jnp.int32(0))

        plsc.subcore_barrier()
        pltpu.sync_copy(sc["acc_sh"].at[pl.ds(sid * rows_per_tile, rows_per_tile)],
                        out_hbm.at[cid, pl.ds(sid * rows_per_tile, rows_per_tile)])
        pltpu.sync_copy(sc["den_loc"], den_hbm.at[cid, sid])

    return pl.kernel(
        body,
        out_type=(jax.ShapeDtypeStruct((NC, n_dst, AW), jnp.float32),
                  jax.ShapeDtypeStruct((NC, NS, DR, 128), jnp.float32)),
        mesh=mesh,
        scratch_types=list(scratch.values()),
        compiler_params=pltpu.CompilerParams(
            needs_layout_passes=False,
            internal_scratch_in_bytes=1024 * 1024,
        ),
    )


_gat_edges = _build_edge_aggregate(10000, 5120, 20000, 80, True)
_pool_edges = _build_edge_aggregate(10000, 5120, 1920, 80, False)


def _prep_body(x_ref, w_ref, aw_ref, out_h, out_asrc, out_adst, out_m):
    h = jnp.dot(x_ref[...], w_ref[...], preferred_element_type=jnp.float32)
    out_h[...] = h
    av = jnp.dot(h, aw_ref[...], preferred_element_type=jnp.float32)
    asrc = av[:, 0]
    adst = av[:, 1]
    out_asrc[...] = asrc
    out_adst[...] = adst
    astar = jnp.max(asrc)
    mu = astar + adst
    out_m[...] = jnp.where(mu >= 0, mu, mu * jnp.float32(0.2))


def _prep(x, W, att_src, att_dst):
    n = x.shape[0]
    aw = jnp.stack([att_src, att_dst], axis=1)
    return pl.pallas_call(
        _prep_body,
        out_shape=(
            jax.ShapeDtypeStruct((n, 128), jnp.float32),
            jax.ShapeDtypeStruct((n,), jnp.float32),
            jax.ShapeDtypeStruct((n,), jnp.float32),
            jax.ShapeDtypeStruct((n,), jnp.float32),
        ),
    )(x, W, aw)


def _finish_body(acc_ref, den_ref, b_ref, out_ref):
    acc = jnp.concatenate([acc_ref[0][:5056, :], acc_ref[1][:4944, :]], axis=0)
    denom = jnp.maximum(den_ref[...], jnp.float32(1e-30))
    x = acc / denom + b_ref[...][None, :]
    out_ref[...] = jnp.maximum(x, jnp.float32(0.0))


def _finish(acc, den2d, b):
    n = 10000
    return pl.pallas_call(
        _finish_body,
        out_shape=jax.ShapeDtypeStruct((n, 128), jnp.float32),
    )(acc, den2d, b)


def _knn_body(hc_ref, hr_ref, out_ref, *, br, bc, n_cols):
    hcb = hc_ref[...]
    cn = jnp.sum(hcb * hcb, axis=1)
    big = jnp.float32(jnp.finfo(jnp.float32).max)

    def col_step(j, carry):
        v0, v1, v2, i0, i1, i2 = carry
        hrb = hr_ref[pl.ds(j * bc, bc), :]
        rn = jnp.sum(hrb * hrb, axis=1)
        ab = lax.dot_general(hcb, hrb, (((1,), (1,)), ((), ())),
                             preferred_element_type=jnp.float32)
        d2 = (cn[:, None] + rn[None, :]) - 2.0 * ab
        colid = lax.broadcasted_iota(jnp.int32, (br, bc), 1) + j * bc
        for _ in range(3):
            c = jnp.min(d2, axis=1)
            am = jnp.min(jnp.where(d2 == c[:, None], colid, jnp.int32(2**30)),
                         axis=1)
            d2 = jnp.where(colid == am[:, None], big, d2)
            b0 = c < v0
            b1 = c < v1
            b2 = c < v2
            nv0 = jnp.where(b0, c, v0)
            nv1 = jnp.where(b0, v0, jnp.where(b1, c, v1))
            nv2 = jnp.where(b1, v1, jnp.where(b2, c, v2))
            ni0 = jnp.where(b0, am, i0)
            ni1 = jnp.where(b0, i0, jnp.where(b1, am, i1))
            ni2 = jnp.where(b1, i1, jnp.where(b2, am, i2))
            v0, v1, v2, i0, i1, i2 = nv0, nv1, nv2, ni0, ni1, ni2
        return (v0, v1, v2, i0, i1, i2)

    init = (jnp.full((br,), big), jnp.full((br,), big), jnp.full((br,), big),
            jnp.zeros((br,), jnp.int32), jnp.zeros((br,), jnp.int32),
            jnp.zeros((br,), jnp.int32))
    _, _, _, i0, i1, i2 = lax.fori_loop(0, n_cols // bc, col_step, init)
    out_ref[...] = jnp.stack([i0, i1, i2], axis=1)


def _knn_top3(hc, hr):
    ncq, nrr = hc.shape[0], hr.shape[0]
    br, bc = 1000, 1000
    return pl.pallas_call(
        functools.partial(_knn_body, br=br, bc=bc, n_cols=nrr),
        grid=(ncq // br,),
        in_specs=[
            pl.BlockSpec((br, 128), lambda i: (i, 0)),
            pl.BlockSpec((nrr, 128), lambda i: (0, 0)),
        ],
        out_specs=pl.BlockSpec((br, 3), lambda i: (i, 0)),
        out_shape=jax.ShapeDtypeStruct((ncq, 3), jnp.int32),
    )(hc, hr)


def _head_body(hr_ref, accp_ref, cnt_ref, wd_ref, bd_ref, out_ref):
    acc = jnp.concatenate([accp_ref[0][:5056, :], accp_ref[1][:4944, :]], axis=0)
    cnt = jnp.maximum(cnt_ref[...], jnp.float32(1.0))
    pooled = acc / cnt
    out_ref[...] = (
        jnp.dot(hr_ref[...], wd_ref[:128, :], preferred_element_type=jnp.float32)
        + jnp.dot(pooled, wd_ref[128:, :], preferred_element_type=jnp.float32)
        + bd_ref[...][None, :])


def _head(hr, accp, cnt2d, Wd, bd):
    n = hr.shape[0]
    return pl.pallas_call(
        _head_body,
        out_shape=jax.ShapeDtypeStruct((n, Wd.shape[1]), jnp.float32),
    )(hr, accp, cnt2d, Wd, bd)


def _den_col(den):
    return jnp.sum(den[0], axis=0).reshape(-1)[:10000].reshape(10000, 1)


def _gat_layer(x, src, dst, zeros, W, att_src, att_dst, b):
    h, asrc, adst, m = _prep(x, W, att_src, att_dst)
    acc, den = _gat_edges(h, asrc, adst, m, src, dst, zeros)
    return _finish(acc, _den_col(den), b), h


def kernel(x_resting, edge_index_resting, x_collider, edge_index_collider,
           W1r, as1r, ad1r, b1r, W2r, as2r, ad2r, b2r,
           W1c, as1c, ad1c, b1c, W2c, as2c, ad2c, b2c, Wd, bd):
    nr = x_resting.shape[0]
    nc = x_collider.shape[0]
    zeros = jnp.zeros((DR, AW), jnp.float32)

    src_r, dst_r = edge_index_resting[0], edge_index_resting[1]
    src_c, dst_c = edge_index_collider[0], edge_index_collider[1]

    x1r, _ = _gat_layer(x_resting, src_r, dst_r, zeros, W1r, as1r, ad1r, b1r)
    hr, _ = _gat_layer(x1r, src_r, dst_r, zeros, W2r, as2r, ad2r, b2r)
    x1c, _ = _gat_layer(x_collider, src_c, dst_c, zeros, W1c, as1c, ad1c, b1c)
    hc, _ = _gat_layer(x1c, src_c, dst_c, zeros, W2c, as2c, ad2c, b2c)

    nn_idx = _knn_top3(hc, hr)
    n_pad = 30720 - nc * 3
    pool_dst = jnp.concatenate(
        [nn_idx.reshape(-1), jnp.full((n_pad,), nr, jnp.int32)])
    pool_src = jnp.concatenate(
        [jnp.repeat(jnp.arange(nc, dtype=jnp.int32), 3),
         jnp.zeros((n_pad,), jnp.int32)])
    accp, denp = _pool_edges(hc, pool_src, pool_dst, zeros)
    return _head(hr, accp, _den_col(denp), Wd, bd)
